# baseline (speedup 1.0000x reference)
import functools

import jax
import jax.numpy as jnp
from jax import lax
from jax.experimental import pallas as pl
from jax.experimental.pallas import tpu as pltpu
from jax.experimental.pallas import tpu_sc as plsc

NUM_CORES = 1
NUM_SUBCORES = 16
NUM_TILES = NUM_CORES * NUM_SUBCORES
LANES = 16


def _make_hist(nbins: int, per_tile: int):
    mesh = plsc.VectorSubcoreMesh(core_axis_name="c", subcore_axis_name="s",
                                  num_cores=NUM_CORES)

    @functools.partial(
        pl.kernel,
        mesh=mesh,
        out_type=jax.ShapeDtypeStruct((NUM_TILES, nbins), jnp.float32),
        scratch_types=[
            pltpu.VMEM((per_tile,), jnp.int32),
            pltpu.VMEM((nbins,), jnp.float32),
            pltpu.SemaphoreType.DMA,
        ],
        compiler_params=pltpu.CompilerParams(needs_layout_passes=False),
    )
    def hist(idx_hbm, out_hbm, idx_v, counts_v, stage_sem):
        c = lax.axis_index("c")
        s = lax.axis_index("s")
        wid = s * NUM_CORES + c

        stage = pltpu.make_async_copy(
            idx_hbm.at[pl.ds(wid * per_tile, per_tile)], idx_v, stage_sem)
        stage.start()

        zu = 25
        assert nbins % (LANES * zu) == 0

        def zbody(i, carry):
            for u in range(zu):
                counts_v[pl.ds((i * zu + u) * LANES, LANES)] = (
                    jnp.zeros((LANES,), jnp.float32))
            return carry
        lax.fori_loop(0, nbins // (LANES * zu), zbody, 0)

        stage.wait()

        su = 25
        assert per_tile % (LANES * su) == 0

        def body(i, carry):
            idxs = [idx_v[pl.ds((i * su + u) * LANES, LANES)]
                    for u in range(su)]
            for idx16 in idxs:
                plsc.addupdate_scatter(
                    counts_v, [idx16], jnp.ones((LANES,), jnp.float32))
            return carry

        lax.fori_loop(0, per_tile // (LANES * su), body, 0)

        pltpu.sync_copy(counts_v, out_hbm.at[wid])

    return hist


def _matvec_body(scale, nchunks, p_ref, z_ref, o_ref, bufs, sems):
    nbins = p_ref.shape[1]
    cb = nbins // nchunks
    cps = []
    for k in range(nchunks):
        cp = pltpu.make_async_copy(
            z_ref.at[pl.ds(k * cb, cb)], bufs[k], sems[k])
        cp.start()
        cps.append(cp)

    counts = jnp.sum(p_ref[...], axis=0, keepdims=True) * scale

    acc = jnp.zeros(o_ref.shape, jnp.float32)
    for k in range(nchunks):
        cps[k].wait()
        acc += lax.dot_general(
            counts[:, k * cb:(k + 1) * cb], bufs[k][...],
            (((1,), (0,)), ((), ())),
            preferred_element_type=jnp.float32,
            precision=lax.Precision.DEFAULT,
        )
    o_ref[...] = acc


def kernel(z, visited_seq):
    nbins, d = z.shape
    n = visited_seq.shape[0]
    assert n % (NUM_TILES * LANES) == 0
    per_tile = n // NUM_TILES

    idx = visited_seq.astype(jnp.int32)
    partials = _make_hist(nbins, per_tile)(idx)

    nchunks = 4
    assert nbins % nchunks == 0
    out = pl.pallas_call(
        functools.partial(_matvec_body, 1.0 / n, nchunks),
        in_specs=[
            pl.BlockSpec(memory_space=pltpu.VMEM),
            pl.BlockSpec(memory_space=pl.ANY),
        ],
        out_shape=jax.ShapeDtypeStruct((1, d), jnp.float32),
        scratch_shapes=[
            [pltpu.VMEM((nbins // nchunks, d), jnp.float32)] * nchunks,
            [pltpu.SemaphoreType.DMA] * nchunks,
        ],
    )(partials, z)
    return out


# speedup vs baseline: 1.0396x; 1.0396x over previous
import functools

import jax
import jax.numpy as jnp
from jax import lax
from jax.experimental import pallas as pl
from jax.experimental.pallas import tpu as pltpu
from jax.experimental.pallas import tpu_sc as plsc

NUM_CORES = 1
NUM_SUBCORES = 16
NUM_TILES = NUM_CORES * NUM_SUBCORES
LANES = 16


def _make_hist(nbins: int, per_tile: int):
    mesh = plsc.VectorSubcoreMesh(core_axis_name="c", subcore_axis_name="s",
                                  num_cores=NUM_CORES)

    @functools.partial(
        pl.kernel,
        mesh=mesh,
        out_type=jax.ShapeDtypeStruct((NUM_TILES, nbins), jnp.float32),
        scratch_types=[
            pltpu.VMEM((per_tile,), jnp.int32),
            pltpu.VMEM((nbins,), jnp.float32),
            pltpu.SemaphoreType.DMA,
        ],
        compiler_params=pltpu.CompilerParams(needs_layout_passes=False),
    )
    def hist(idx_hbm, out_hbm, idx_v, counts_v, stage_sem):
        c = lax.axis_index("c")
        s = lax.axis_index("s")
        wid = s * NUM_CORES + c

        stage = pltpu.make_async_copy(
            idx_hbm.at[pl.ds(wid * per_tile, per_tile)], idx_v, stage_sem)
        stage.start()

        zu = 25
        assert nbins % (LANES * zu) == 0

        def zbody(i, carry):
            for u in range(zu):
                counts_v[pl.ds((i * zu + u) * LANES, LANES)] = (
                    jnp.zeros((LANES,), jnp.float32))
            return carry
        lax.fori_loop(0, nbins // (LANES * zu), zbody, 0)

        stage.wait()

        su = 25
        assert per_tile % (LANES * su) == 0

        def body(i, carry):
            idxs = [idx_v[pl.ds((i * su + u) * LANES, LANES)]
                    for u in range(su)]
            for idx16 in idxs:
                plsc.addupdate_scatter(
                    counts_v, [idx16], jnp.ones((LANES,), jnp.float32))
            return carry

        lax.fori_loop(0, per_tile // (LANES * su), body, 0)

        pltpu.sync_copy(counts_v, out_hbm.at[wid])

    return hist


def _matvec_body(scale, p_ref, z_ref, o_ref):
    counts = jnp.sum(p_ref[...], axis=0, keepdims=True) * scale
    o_ref[...] = lax.dot_general(
        counts, z_ref[...], (((1,), (0,)), ((), ())),
        preferred_element_type=jnp.float32,
        precision=lax.Precision.DEFAULT,
    )


def kernel(z, visited_seq):
    nbins, d = z.shape
    n = visited_seq.shape[0]
    assert n % (NUM_TILES * LANES) == 0
    per_tile = n // NUM_TILES

    idx = visited_seq.astype(jnp.int32)
    partials = _make_hist(nbins, per_tile)(idx)

    out = pl.pallas_call(
        functools.partial(_matvec_body, 1.0 / n),
        out_shape=jax.ShapeDtypeStruct((1, d), jnp.float32),
    )(partials, z)
    return out
